# baseline (speedup 1.0000x reference)
import functools

import jax
import jax.numpy as jnp
from jax import lax
from jax.experimental import pallas as pl
from jax.experimental.pallas import tpu as pltpu
from jax.experimental.pallas import tpu_sc as plsc

N = 10000
K = 27
E_PER = 5925
E = K * E_PER
E_PAD = 163840
N_SUB = 16
CHUNK = 128
CHUNKS = E_PAD // (N_SUB * CHUNK)
R = K * N
ACC_ROWS = 10240
NBUF = 8
LOOKAHEAD = 4
BN = 1000
NB = N // BN


def _mm_conv(x2, aff, w, relu, p_split):
    cin, cout = w.shape[1], w.shape[2]
    chi = cin // 2
    cw = cout // (2 * p_split)

    def body(x_ref, a_ref, w_ref, *h_refs):
        k = pl.program_id(1)
        sc = a_ref[:, 0]
        sh = a_ref[:, 1]
        act = x_ref[...] * sc[:, None, :] + sh[:, None, :]
        if relu:
            act = jnp.maximum(act, 0.0)
        xcat = jnp.concatenate([act[0], act[1]], axis=-1)
        wk = w_ref[pl.ds(k, 1)][0]
        h = jnp.dot(xcat, wk, preferred_element_type=jnp.float32)
        parts = h.reshape(BN, 2, p_split, cw)
        for p in range(p_split):
            h_refs[p][...] = parts[:, :, p].transpose(1, 0, 2)

    return pl.pallas_call(
        body,
        grid=(NB, K),
        in_specs=[
            pl.BlockSpec((2, BN, chi), lambda nb, k: (0, nb, 0)),
            pl.BlockSpec((2, 2, chi), lambda nb, k: (0, 0, 0)),
            pl.BlockSpec((K, cin, cout), lambda nb, k: (0, 0, 0)),
        ],
        out_specs=[
            pl.BlockSpec((2, BN, cw), lambda nb, k: (0, k * NB + nb, 0))
            for _ in range(p_split)
        ],
        out_shape=[
            jax.ShapeDtypeStruct((2, R, cw), jnp.float32)
            for _ in range(p_split)
        ],
    )(x2, aff, w)


def _stats(y2, g2, b2):
    chh = y2.shape[2]

    def body(y_ref, g_ref, b_ref, o_ref):
        yb = y_ref[:, :N]
        mean = jnp.mean(yb, axis=1)
        var = jnp.mean(yb * yb, axis=1) - mean * mean
        scale = g_ref[...] * lax.rsqrt(var + 1e-5)
        o_ref[...] = jnp.stack([scale, b_ref[...] - mean * scale], axis=1)

    return pl.pallas_call(
        body,
        out_shape=jax.ShapeDtypeStruct((2, 2, chh), jnp.float32),
    )(y2, g2, b2)


def _mm_dense2(a2, affa, b2, affb, w):
    cha, chb = a2.shape[2], b2.shape[2]
    cout = w.shape[1]
    cho = cout // 2

    def body(a_ref, aa_ref, b_ref, ab_ref, w_ref, o_ref):
        aa = jnp.maximum(a_ref[...] * aa_ref[:, 0][:, None, :]
                         + aa_ref[:, 1][:, None, :], 0.0)
        ab = jnp.maximum(b_ref[...] * ab_ref[:, 0][:, None, :]
                         + ab_ref[:, 1][:, None, :], 0.0)
        xcat = jnp.concatenate([aa[0], aa[1], ab[0], ab[1]], axis=-1)
        h = jnp.dot(xcat, w_ref[...], preferred_element_type=jnp.float32)
        o_ref[...] = jnp.stack([h[:, :cho], h[:, cho:]], axis=0)

    return pl.pallas_call(
        body,
        grid=(NB,),
        in_specs=[
            pl.BlockSpec((2, BN, cha), lambda nb: (0, nb, 0)),
            pl.BlockSpec((2, 2, cha), lambda nb: (0, 0, 0)),
            pl.BlockSpec((2, BN, chb), lambda nb: (0, nb, 0)),
            pl.BlockSpec((2, 2, chb), lambda nb: (0, 0, 0)),
            pl.BlockSpec(w.shape, lambda nb: (0, 0)),
        ],
        out_specs=pl.BlockSpec((2, BN, cho), lambda nb: (0, nb, 0)),
        out_shape=jax.ShapeDtypeStruct((2, N, cho), jnp.float32),
    )(a2, affa, b2, affb, w)


def _mm_final(a2, affa, w):
    cha = a2.shape[2]
    cout = w.shape[1]

    def body(a_ref, aa_ref, w_ref, o_ref):
        aa = jnp.maximum(a_ref[...] * aa_ref[:, 0][:, None, :]
                         + aa_ref[:, 1][:, None, :], 0.0)
        xcat = jnp.concatenate([aa[0], aa[1]], axis=-1)
        o_ref[...] = jnp.dot(xcat, w_ref[...],
                             preferred_element_type=jnp.float32)

    return pl.pallas_call(
        body,
        grid=(NB,),
        in_specs=[
            pl.BlockSpec((2, BN, cha), lambda nb: (0, nb, 0)),
            pl.BlockSpec((2, 2, cha), lambda nb: (0, 0, 0)),
            pl.BlockSpec(w.shape, lambda nb: (0, 0)),
        ],
        out_specs=pl.BlockSpec((BN, cout), lambda nb: (nb, 0)),
        out_shape=jax.ShapeDtypeStruct((N, cout), jnp.float32),
    )(a2, affa, w)


@functools.lru_cache(maxsize=None)
def _make_sc_conv(cho):
    mesh = plsc.VectorSubcoreMesh(core_axis_name="c", subcore_axis_name="s")

    @functools.partial(
        pl.kernel,
        mesh=mesh,
        compiler_params=pltpu.CompilerParams(use_tc_tiling_on_sc=False),
        out_type=jax.ShapeDtypeStruct((2, ACC_ROWS, cho), jnp.float32),
        scratch_types=[
            pltpu.VMEM((CHUNKS, CHUNK), jnp.int32),
            pltpu.VMEM((CHUNKS, CHUNK), jnp.int32),
            [pltpu.VMEM((CHUNK, cho), jnp.float32) for _ in range(NBUF)],
            pltpu.VMEM_SHARED((ACC_ROWS, cho), jnp.float32),
            [pltpu.SemaphoreType.DMA for _ in range(NBUF)],
            [pltpu.SemaphoreType.DMA for _ in range(NBUF)],
        ],
    )
    def sc_conv(h_hbm, gidx_hbm, dst_hbm, y_hbm, gv, dv, rows, acc,
                gsems, ssems):
        zbuf = rows[0]
        c = lax.axis_index("c")
        s = lax.axis_index("s")
        zvec = jnp.zeros((16,), jnp.float32)

        def zrow(r, carry):
            for i in range(cho // 16):
                zbuf[r, pl.ds(i * 16, 16)] = zvec
            return carry

        lax.fori_loop(0, CHUNK, zrow, 0)
        for t in range(ACC_ROWS // (N_SUB * CHUNK)):
            pltpu.sync_copy(zbuf, acc.at[pl.ds(s * 640 + t * CHUNK, CHUNK)])
        pltpu.sync_copy(gidx_hbm.at[c, s], gv)
        pltpu.sync_copy(dst_hbm.at[s], dv)
        plsc.subcore_barrier()

        for b in range(LOOKAHEAD):
            pltpu.async_copy(h_hbm.at[gv.at[b]], rows[b], gsems[b])

        def step(g, carry):
            for i in range(NBUF):
                j = g * NBUF + i
                b = i
                pltpu.make_async_copy(h_hbm.at[gv.at[j]], rows[b],
                                      gsems[b]).wait()
                pltpu.async_copy(rows[b], acc.at[dv.at[j]], ssems[b],
                                 add=True)
                jd = j - LOOKAHEAD
                bd = (i - LOOKAHEAD) % NBUF

                @pl.when(jd >= 0)
                def _():
                    pltpu.make_async_copy(
                        rows[bd], acc.at[dv.at[jd]], ssems[bd]).wait()

                jn = j + LOOKAHEAD
                bn = (i + LOOKAHEAD) % NBUF

                @pl.when(jn < CHUNKS)
                def _():
                    pltpu.async_copy(h_hbm.at[gv.at[jn]], rows[bn], gsems[bn])
            return carry

        lax.fori_loop(0, CHUNKS // NBUF, step, 0)
        for i in range(LOOKAHEAD):
            j = CHUNKS - LOOKAHEAD + i
            pltpu.make_async_copy(rows[j % NBUF], acc.at[dv.at[j]],
                                  ssems[j % NBUF]).wait()
        plsc.subcore_barrier()
        pltpu.sync_copy(acc.at[pl.ds(s * 640, 640)],
                        y_hbm.at[c, pl.ds(s * 640, 640)])

    return sc_conv


def _sc_conv(h2, gidx2, dst2):
    cw = h2.shape[2]
    return _make_sc_conv(cw)(h2.reshape(2 * R, cw), gidx2, dst2)


def _g2(v):
    return v.reshape(2, v.shape[0] // 2)


def _conv(x2, aff, relu, w, gidx2, dst2):
    cho = w.shape[2] // 2
    p_split = max(1, cho // 64)
    hs = _mm_conv(x2, aff, w, relu, p_split)
    ys = [_sc_conv(h, gidx2, dst2) for h in hs]
    return ys[0] if p_split == 1 else jnp.concatenate(ys, axis=2)


def _block(x2, aff_in, relu_in, p, gidx2, dst2):
    y1 = _conv(x2, aff_in, relu_in, p["W1"], gidx2, dst2)
    aff1 = _stats(y1, _g2(p["g1"]), _g2(p["b1"]))
    y2 = _conv(y1, aff1, True, p["W2"], gidx2, dst2)
    aff2 = _stats(y2, _g2(p["g2"]), _g2(p["b2"]))
    return y2, aff2


def kernel(x, edge_index, params):
    src = edge_index[0].astype(jnp.int32)
    dst = edge_index[1].astype(jnp.int32)
    gidx = (jnp.arange(E, dtype=jnp.int32) // E_PER) * N + src
    gidx = jnp.concatenate([gidx, jnp.zeros((E_PAD - E,), jnp.int32)])
    dstp = jnp.concatenate([dst, jnp.full((E_PAD - E,), N, jnp.int32)])
    gidx2 = jnp.stack([gidx, gidx + R]).reshape(2, N_SUB, CHUNKS, CHUNK)
    dst2 = dstp.reshape(N_SUB, CHUNKS, CHUNK)

    x2 = jnp.moveaxis(x.reshape(N, 2, 64), 1, 0)
    one = jnp.ones((2, 64), jnp.float32)
    aff0 = jnp.stack([one, jnp.zeros_like(one)], axis=1)

    y_e0, aff_e0 = _block(x2, aff0, False, params["enc0"], gidx2, dst2)
    y_e1, aff_e1 = _block(y_e0, aff_e0, True, params["enc1"], gidx2, dst2)
    y_bt, aff_bt = _block(y_e1, aff_e1, True, params["bottleneck"], gidx2, dst2)

    d0 = _mm_dense2(y_bt, aff_bt, y_e1, aff_e1, params["dec0"]["Wf"])
    aff_d0 = _stats(d0, _g2(params["dec0"]["g"]), _g2(params["dec0"]["b"]))
    d1 = _mm_dense2(d0, aff_d0, y_e0, aff_e0, params["dec1"]["Wf"])
    aff_d1 = _stats(d1, _g2(params["dec1"]["g"]), _g2(params["dec1"]["b"]))
    return _mm_final(d1, aff_d1, params["final_W"])


# speedup vs baseline: 1.8325x; 1.8325x over previous
import functools

import jax
import jax.numpy as jnp
from jax import lax
from jax.experimental import pallas as pl
from jax.experimental.pallas import tpu as pltpu
from jax.experimental.pallas import tpu_sc as plsc

N = 10000
K = 27
E_PER = 5925
E = K * E_PER
E_PAD = 163840
N_SUB = 16
CHUNK = 128
CHUNKS = E_PAD // (N_SUB * CHUNK)
R = K * N
ACC_ROWS = 10240
NBUF = 8
LOOKAHEAD = 4
BN = 1000
NB = N // BN
BN2 = 200


def _mm_conv(x2, aff, w, relu, p_split):
    cin, cout = w.shape[1], w.shape[2]
    chi = cin // 2
    cw = cout // (2 * p_split)
    wr = (w.transpose(1, 0, 2)
          .reshape(cin, K, 2 * p_split, cw)
          .transpose(0, 2, 1, 3)
          .reshape(cin, 2 * p_split * K * cw)
          .astype(jnp.bfloat16))

    ncols = 2 * p_split * K * cw

    def body(x_ref, a_ref, w_ref, h_ref):
        sc = a_ref[:, 0]
        sh = a_ref[:, 1]
        act = x_ref[...] * sc[:, None, :] + sh[:, None, :]
        if relu:
            act = jnp.maximum(act, 0.0)
        xcat = jnp.concatenate([act[0], act[1]], axis=-1).astype(jnp.bfloat16)
        h_ref[...] = jnp.dot(xcat, w_ref[...],
                             preferred_element_type=jnp.float32)

    out = pl.pallas_call(
        body,
        grid=(N // BN2,),
        in_specs=[
            pl.BlockSpec((2, BN2, chi), lambda nb: (0, nb, 0)),
            pl.BlockSpec((2, 2, chi), lambda nb: (0, 0, 0)),
            pl.BlockSpec((cin, ncols), lambda nb: (0, 0)),
        ],
        out_specs=pl.BlockSpec((BN2, ncols), lambda nb: (nb, 0)),
        out_shape=jax.ShapeDtypeStruct((N, ncols), jnp.float32),
    )(x2, aff, wr)
    return out.reshape(N * 2 * p_split * K, cw)


def _stats(y2, g2, b2):
    chh = y2.shape[2]

    def body(y_ref, g_ref, b_ref, o_ref):
        yb = y_ref[:, :N]
        mean = jnp.mean(yb, axis=1)
        var = jnp.mean(yb * yb, axis=1) - mean * mean
        scale = g_ref[...] * lax.rsqrt(var + 1e-5)
        o_ref[...] = jnp.stack([scale, b_ref[...] - mean * scale], axis=1)

    return pl.pallas_call(
        body,
        out_shape=jax.ShapeDtypeStruct((2, 2, chh), jnp.float32),
    )(y2, g2, b2)


def _mm_dense2(a2, affa, b2, affb, w):
    cha, chb = a2.shape[2], b2.shape[2]
    cout = w.shape[1]
    cho = cout // 2

    def body(a_ref, aa_ref, b_ref, ab_ref, w_ref, o_ref):
        aa = jnp.maximum(a_ref[...] * aa_ref[:, 0][:, None, :]
                         + aa_ref[:, 1][:, None, :], 0.0)
        ab = jnp.maximum(b_ref[...] * ab_ref[:, 0][:, None, :]
                         + ab_ref[:, 1][:, None, :], 0.0)
        xcat = jnp.concatenate([aa[0], aa[1], ab[0], ab[1]], axis=-1)
        h = jnp.dot(xcat, w_ref[...], preferred_element_type=jnp.float32)
        o_ref[...] = jnp.stack([h[:, :cho], h[:, cho:]], axis=0)

    return pl.pallas_call(
        body,
        grid=(NB,),
        in_specs=[
            pl.BlockSpec((2, BN, cha), lambda nb: (0, nb, 0)),
            pl.BlockSpec((2, 2, cha), lambda nb: (0, 0, 0)),
            pl.BlockSpec((2, BN, chb), lambda nb: (0, nb, 0)),
            pl.BlockSpec((2, 2, chb), lambda nb: (0, 0, 0)),
            pl.BlockSpec(w.shape, lambda nb: (0, 0)),
        ],
        out_specs=pl.BlockSpec((2, BN, cho), lambda nb: (0, nb, 0)),
        out_shape=jax.ShapeDtypeStruct((2, N, cho), jnp.float32),
    )(a2, affa, b2, affb, w)


def _mm_final(a2, affa, w):
    cha = a2.shape[2]
    cout = w.shape[1]

    def body(a_ref, aa_ref, w_ref, o_ref):
        aa = jnp.maximum(a_ref[...] * aa_ref[:, 0][:, None, :]
                         + aa_ref[:, 1][:, None, :], 0.0)
        xcat = jnp.concatenate([aa[0], aa[1]], axis=-1)
        o_ref[...] = jnp.dot(xcat, w_ref[...],
                             preferred_element_type=jnp.float32)

    return pl.pallas_call(
        body,
        grid=(NB,),
        in_specs=[
            pl.BlockSpec((2, BN, cha), lambda nb: (0, nb, 0)),
            pl.BlockSpec((2, 2, cha), lambda nb: (0, 0, 0)),
            pl.BlockSpec(w.shape, lambda nb: (0, 0)),
        ],
        out_specs=pl.BlockSpec((BN, cout), lambda nb: (nb, 0)),
        out_shape=jax.ShapeDtypeStruct((N, cout), jnp.float32),
    )(a2, affa, w)


@functools.lru_cache(maxsize=None)
def _make_sc_conv(cho):
    mesh = plsc.VectorSubcoreMesh(core_axis_name="c", subcore_axis_name="s")

    @functools.partial(
        pl.kernel,
        mesh=mesh,
        compiler_params=pltpu.CompilerParams(use_tc_tiling_on_sc=False),
        out_type=jax.ShapeDtypeStruct((2, ACC_ROWS, cho), jnp.float32),
        scratch_types=[
            pltpu.VMEM((CHUNKS, CHUNK), jnp.int32),
            pltpu.VMEM((CHUNKS, CHUNK), jnp.int32),
            [pltpu.VMEM((CHUNK, cho), jnp.float32) for _ in range(NBUF)],
            pltpu.VMEM_SHARED((ACC_ROWS, cho), jnp.float32),
            [pltpu.SemaphoreType.DMA for _ in range(NBUF)],
            [pltpu.SemaphoreType.DMA for _ in range(NBUF)],
        ],
    )
    def sc_conv(h_hbm, gidx_hbm, dst_hbm, y_hbm, gv, dv, rows, acc,
                gsems, ssems):
        zbuf = rows[0]
        c = lax.axis_index("c")
        s = lax.axis_index("s")
        zvec = jnp.zeros((16,), jnp.float32)

        def zrow(r, carry):
            for i in range(cho // 16):
                zbuf[r, pl.ds(i * 16, 16)] = zvec
            return carry

        lax.fori_loop(0, CHUNK, zrow, 0)
        for t in range(ACC_ROWS // (N_SUB * CHUNK)):
            pltpu.sync_copy(zbuf, acc.at[pl.ds(s * 640 + t * CHUNK, CHUNK)])
        pltpu.sync_copy(gidx_hbm.at[c, s], gv)
        pltpu.sync_copy(dst_hbm.at[s], dv)
        plsc.subcore_barrier()

        for b in range(LOOKAHEAD):
            pltpu.async_copy(h_hbm.at[gv.at[b]], rows[b], gsems[b])

        def step(g, carry):
            for i in range(NBUF):
                j = g * NBUF + i
                b = i
                pltpu.make_async_copy(h_hbm.at[gv.at[j]], rows[b],
                                      gsems[b]).wait()
                pltpu.async_copy(rows[b], acc.at[dv.at[j]], ssems[b],
                                 add=True)
                jd = j - LOOKAHEAD
                bd = (i - LOOKAHEAD) % NBUF

                @pl.when(jd >= 0)
                def _():
                    pltpu.make_async_copy(
                        rows[bd], acc.at[dv.at[jd]], ssems[bd]).wait()

                jn = j + LOOKAHEAD
                bn = (i + LOOKAHEAD) % NBUF

                @pl.when(jn < CHUNKS)
                def _():
                    pltpu.async_copy(h_hbm.at[gv.at[jn]], rows[bn], gsems[bn])
            return carry

        lax.fori_loop(0, CHUNKS // NBUF, step, 0)
        for i in range(LOOKAHEAD):
            j = CHUNKS - LOOKAHEAD + i
            pltpu.make_async_copy(rows[j % NBUF], acc.at[dv.at[j]],
                                  ssems[j % NBUF]).wait()
        plsc.subcore_barrier()
        pltpu.sync_copy(acc.at[pl.ds(s * 640, 640)],
                        y_hbm.at[c, pl.ds(s * 640, 640)])

    return sc_conv


def _sc_conv(h_flat, gidx2, dst2):
    return _make_sc_conv(h_flat.shape[1])(h_flat, gidx2, dst2)


def _g2(v):
    return v.reshape(2, v.shape[0] // 2)


def _conv(x2, aff, relu, w, gidx_list, dst2):
    cho = w.shape[2] // 2
    p_split = max(1, cho // 64)
    h_flat = _mm_conv(x2, aff, w, relu, p_split)
    ys = [_sc_conv(h_flat, gidx_list[p_split][p], dst2)
          for p in range(p_split)]
    return ys[0] if p_split == 1 else jnp.concatenate(ys, axis=2)


def _block(x2, aff_in, relu_in, p, gidx_list, dst2):
    y1 = _conv(x2, aff_in, relu_in, p["W1"], gidx_list, dst2)
    aff1 = _stats(y1, _g2(p["g1"]), _g2(p["b1"]))
    y2 = _conv(y1, aff1, True, p["W2"], gidx_list, dst2)
    aff2 = _stats(y2, _g2(p["g2"]), _g2(p["b2"]))
    return y2, aff2


def kernel(x, edge_index, params):
    src = edge_index[0].astype(jnp.int32)
    dst = edge_index[1].astype(jnp.int32)
    k_of = jnp.arange(E, dtype=jnp.int32) // E_PER
    gidx_list = {}
    for p_split in (1, 2):
        base = src * (2 * p_split * K) + k_of
        base = jnp.concatenate([base, jnp.zeros((E_PAD - E,), jnp.int32)])
        gidx_list[p_split] = [
            jnp.stack([base + (0 * p_split + p) * K,
                       base + (1 * p_split + p) * K]
                      ).reshape(2, N_SUB, CHUNKS, CHUNK)
            for p in range(p_split)
        ]
    dstp = jnp.concatenate([dst, jnp.full((E_PAD - E,), N, jnp.int32)])
    dst2 = dstp.reshape(N_SUB, CHUNKS, CHUNK)

    x2 = jnp.moveaxis(x.reshape(N, 2, 64), 1, 0)
    one = jnp.ones((2, 64), jnp.float32)
    aff0 = jnp.stack([one, jnp.zeros_like(one)], axis=1)

    y_e0, aff_e0 = _block(x2, aff0, False, params["enc0"], gidx_list, dst2)
    y_e1, aff_e1 = _block(y_e0, aff_e0, True, params["enc1"], gidx_list, dst2)
    y_bt, aff_bt = _block(y_e1, aff_e1, True, params["bottleneck"], gidx_list, dst2)

    d0 = _mm_dense2(y_bt, aff_bt, y_e1, aff_e1, params["dec0"]["Wf"])
    aff_d0 = _stats(d0, _g2(params["dec0"]["g"]), _g2(params["dec0"]["b"]))
    d1 = _mm_dense2(d0, aff_d0, y_e0, aff_e0, params["dec1"]["Wf"])
    aff_d1 = _stats(d1, _g2(params["dec1"]["g"]), _g2(params["dec1"]["b"]))
    return _mm_final(d1, aff_d1, params["final_W"])
